# Initial kernel scaffold; baseline (speedup 1.0000x reference)
import jax
import jax.numpy as jnp
from jax.experimental import pallas as pl


def kernel(user_ids, user_cat_feats, user_numeric_feats, user_emb_table, cat_tables, W1, b1, W2, b2):
    raise NotImplementedError("write your pallas kernel here")


# speedup vs baseline: 6.1831x; 6.1831x over previous
import functools

import jax
import jax.numpy as jnp
from jax import lax
from jax.experimental import pallas as pl
from jax.experimental.pallas import tpu as pltpu
from jax.experimental.pallas import tpu_sc as plsc

B = 16384
NUM_CAT = 26
CAT_V = 100000
CAT_E = 16
ID_E = 32
NUM_NUM = 13
H = 128
OUT = 64

NC, NS = 2, 16
NW = NC * NS
BPW = B // NW
CAT_ROWS_PW = BPW * NUM_CAT
NCHUNK = 8
CHUNK = CAT_ROWS_PW // NCHUNK


@functools.lru_cache(maxsize=None)
def _build_sc_gather():
    @functools.partial(
        pl.kernel,
        out_type=(
            jax.ShapeDtypeStruct((B, ID_E), jnp.float32),
            jax.ShapeDtypeStruct((B * NUM_CAT, CAT_E), jnp.float32),
        ),
        mesh=plsc.VectorSubcoreMesh(
            core_axis_name="c", subcore_axis_name="s",
            num_cores=NC, num_subcores=NS,
        ),
        scratch_types=[
            pltpu.VMEM((BPW,), jnp.int32),
            pltpu.VMEM((BPW, ID_E), jnp.float32),
            pltpu.VMEM((CHUNK,), jnp.int32),
            pltpu.VMEM((CHUNK, CAT_E), jnp.float32),
            pltpu.SemaphoreType.DMA,
        ],
        compiler_params=pltpu.CompilerParams(use_tc_tiling_on_sc=False),
    )
    def _sc_gather(ids_hbm, catidx_hbm, idtab_hbm, cattab_hbm, id_out, cat_out,
                   idx_id, rows_id, idx_c, rows_c, sem):
        wid = lax.axis_index("s") * NC + lax.axis_index("c")
        base = wid * BPW
        pltpu.sync_copy(ids_hbm.at[pl.ds(base, BPW)], idx_id)
        pltpu.async_copy(idtab_hbm.at[idx_id], rows_id, sem).wait()
        pltpu.sync_copy(rows_id, id_out.at[pl.ds(base, BPW)])
        cbase = wid * CAT_ROWS_PW
        for i in range(NCHUNK):
            off = cbase + i * CHUNK
            pltpu.sync_copy(catidx_hbm.at[pl.ds(off, CHUNK)], idx_c)
            pltpu.async_copy(cattab_hbm.at[idx_c], rows_c, sem).wait()
            pltpu.sync_copy(rows_c, cat_out.at[pl.ds(off, CHUNK)])

    return _sc_gather


BM = 2048


def _mlp_body(id_ref, cat_ref, num_ref, w1a_ref, w1b_ref, w1c_ref, b1_ref,
              w2_ref, b2_ref, o_ref):
    h = jnp.dot(cat_ref[...], w1b_ref[...], preferred_element_type=jnp.float32)
    h += jnp.dot(id_ref[...], w1a_ref[...], preferred_element_type=jnp.float32)
    h += jnp.dot(num_ref[...], w1c_ref[...], preferred_element_type=jnp.float32)
    h = jnp.maximum(h + b1_ref[...], 0.0)
    o = jnp.dot(h, w2_ref[...], preferred_element_type=jnp.float32) + b2_ref[...]
    n = jnp.sqrt(jnp.sum(o * o, axis=1, keepdims=True))
    o_ref[...] = o / jnp.maximum(n, 1e-12)


def _mlp(id_emb, cat_emb, num_feats, w1a, w1b, w1c, b1, w2, b2):
    grid = (B // BM,)
    return pl.pallas_call(
        _mlp_body,
        grid=grid,
        in_specs=[
            pl.BlockSpec((BM, ID_E), lambda i: (i, 0)),
            pl.BlockSpec((BM, NUM_CAT * CAT_E), lambda i: (i, 0)),
            pl.BlockSpec((BM, NUM_NUM), lambda i: (i, 0)),
            pl.BlockSpec((ID_E, H), lambda i: (0, 0)),
            pl.BlockSpec((NUM_CAT * CAT_E, H), lambda i: (0, 0)),
            pl.BlockSpec((NUM_NUM, H), lambda i: (0, 0)),
            pl.BlockSpec((1, H), lambda i: (0, 0)),
            pl.BlockSpec((H, OUT), lambda i: (0, 0)),
            pl.BlockSpec((1, OUT), lambda i: (0, 0)),
        ],
        out_specs=pl.BlockSpec((BM, OUT), lambda i: (i, 0)),
        out_shape=jax.ShapeDtypeStruct((B, OUT), jnp.float32),
        compiler_params=pltpu.CompilerParams(
            dimension_semantics=("arbitrary",)),
    )(id_emb, cat_emb, num_feats, w1a, w1b, w1c, b1, w2, b2)


def kernel(user_ids, user_cat_feats, user_numeric_feats, user_emb_table,
           cat_tables, W1, b1, W2, b2):
    catidx = (user_cat_feats.astype(jnp.int32)
              + (jnp.arange(NUM_CAT, dtype=jnp.int32) * CAT_V)[None, :]
              ).reshape(B * NUM_CAT)
    cat_flat = cat_tables.reshape(NUM_CAT * CAT_V, CAT_E)
    id_emb, cat_rows = _build_sc_gather()(user_ids.astype(jnp.int32), catidx,
                                          user_emb_table, cat_flat)
    cat_emb = cat_rows.reshape(B, NUM_CAT * CAT_E)
    w1a = W1[:ID_E]
    w1b = W1[ID_E:ID_E + NUM_CAT * CAT_E]
    w1c = W1[ID_E + NUM_CAT * CAT_E:]
    return _mlp(id_emb, cat_emb, user_numeric_feats,
                w1a, w1b, w1c, b1.reshape(1, H), W2, b2.reshape(1, OUT))
